# baseline (speedup 1.0000x reference)
import functools

import jax
import jax.numpy as jnp
from jax import lax
from jax.experimental import pallas as pl
from jax.experimental.pallas import tpu as pltpu
from jax.experimental.pallas import tpu_sc as plsc

EMBED = 64
FEAT = 3
LANES = 16
NWORKERS = 32
CHUNK = 128
NBUF = 4
NJ = EMBED // LANES


def _sc_call(n_tokens):
    assert n_tokens % (NWORKERS * CHUNK) == 0
    per_worker = n_tokens // NWORKERS
    rows = per_worker // CHUNK
    assert rows % NBUF == 0
    kiters = rows // NBUF

    mesh = plsc.VectorSubcoreMesh(core_axis_name="c", subcore_axis_name="s")

    @functools.partial(
        pl.kernel,
        out_type=jax.ShapeDtypeStruct((n_tokens, EMBED), jnp.float32),
        mesh=mesh,
        compiler_params=pltpu.CompilerParams(use_tc_tiling_on_sc=False),
        scratch_types=[
            pltpu.VMEM((rows, CHUNK), jnp.int32),
            pltpu.VMEM((NBUF * CHUNK * FEAT,), jnp.float32),
            pltpu.VMEM((NBUF, CHUNK, EMBED), jnp.float32),
            pltpu.VMEM((FEAT, EMBED), jnp.float32),
            pltpu.VMEM((EMBED,), jnp.float32),
        ] + [pltpu.SemaphoreType.DMA] * (3 * NBUF),
    )
    def k(ids_hbm, feats_hbm, table_hbm, w_hbm, b_hbm, out_hbm,
          ids_v, feats_v, acc_v, w_v, b_v, *sems):
        gsem = sems[0:NBUF]
        fsem = sems[NBUF:2 * NBUF]
        osem = sems[2 * NBUF:3 * NBUF]
        wid = lax.axis_index("s") * 2 + lax.axis_index("c")
        tok0 = wid * per_worker

        pltpu.sync_copy(w_hbm, w_v)
        pltpu.sync_copy(b_hbm, b_v)
        pltpu.sync_copy(ids_hbm.at[pl.ds(wid * rows, rows)], ids_v)
        wv = [[w_v[r, pl.ds(LANES * j, LANES)] for j in range(NJ)]
              for r in range(FEAT)]
        bv = [b_v[pl.ds(LANES * j, LANES)] for j in range(NJ)]
        iota = lax.iota(jnp.int32, 16)

        FCH = CHUNK * FEAT

        def start_chunk(c, m):
            pltpu.async_copy(table_hbm.at[ids_v.at[c]], acc_v.at[m], gsem[m])
            pltpu.async_copy(feats_hbm.at[pl.ds((tok0 + c * CHUNK) * FEAT, FCH)],
                             feats_v.at[pl.ds(m * FCH, FCH)], fsem[m])

        def wait_chunk(c, m):
            pltpu.make_async_copy(
                table_hbm.at[ids_v.at[c]], acc_v.at[m], gsem[m]).wait()
            pltpu.make_async_copy(
                feats_hbm.at[pl.ds(tok0 * FEAT, FCH)],
                feats_v.at[pl.ds(m * FCH, FCH)], fsem[m]).wait()

        def compute(m):
            @pl.loop(0, CHUNK // 16)
            def _(g):
                fbase = m * (CHUNK * FEAT) + g * (16 * FEAT)
                vq = [feats_v[pl.ds(fbase + q * 16, 16)] for q in range(FEAT)]
                for i in range(16):
                    row = g * 16 + i
                    s = [vq[(FEAT * i + r) // 16][(FEAT * i + r) % 16]
                         for r in range(FEAT)]
                    for j in range(NJ):
                        p = bv[j] + s[0] * wv[0][j]
                        p = p + s[1] * wv[1][j]
                        p = p + s[2] * wv[2][j]
                        plsc.addupdate(
                            acc_v.at[m, row, pl.ds(LANES * j, LANES)], p)

        def start_scatter(c, m):
            dst = out_hbm.at[pl.ds(tok0 + c * CHUNK, CHUNK)]
            pltpu.async_copy(acc_v.at[m], dst, osem[m])

        def wait_scatter(m):
            pltpu.make_async_copy(
                acc_v.at[m], out_hbm.at[pl.ds(tok0, CHUNK)], osem[m]).wait()

        @pl.loop(0, kiters)
        def _(k_it):
            for m in range(NBUF):
                c = k_it * NBUF + m
                mp = (m - 1) % NBUF

                @pl.when(k_it > 0)
                def _():
                    wait_scatter(m)

                start_chunk(c, m)

                def fin():
                    wait_chunk(c - 1, mp)
                    compute(mp)
                    start_scatter(c - 1, mp)

                if m == 0:
                    @pl.when(k_it > 0)
                    def _():
                        fin()
                else:
                    fin()

        last = rows - 1
        wait_chunk(last, last % NBUF)
        compute(last % NBUF)
        start_scatter(last, last % NBUF)
        for m in range(NBUF):
            wait_scatter(m)

    return k


def kernel(ids, feats, table, W, b):
    bsz, seq = ids.shape
    n = bsz * seq
    ids2d = ids.astype(jnp.int32).reshape(n // CHUNK, CHUNK)
    featsf = feats.reshape(n * FEAT)
    out = _sc_call(n)(ids2d, featsf, table, W, b)
    return out.reshape(bsz, seq, EMBED)


# speedup vs baseline: 1.1008x; 1.1008x over previous
import functools

import jax
import jax.numpy as jnp
from jax import lax
from jax.experimental import pallas as pl
from jax.experimental.pallas import tpu as pltpu
from jax.experimental.pallas import tpu_sc as plsc

EMBED = 64
FEAT = 3
LANES = 16
NWORKERS = 32
CHUNK = 128
NBUF = 4
NJ = EMBED // LANES


def _sc_call(n_tokens):
    assert n_tokens % (NWORKERS * CHUNK) == 0
    per_worker = n_tokens // NWORKERS
    rows = per_worker // CHUNK
    assert rows % NBUF == 0
    kiters = rows // NBUF

    mesh = plsc.VectorSubcoreMesh(core_axis_name="c", subcore_axis_name="s")

    @functools.partial(
        pl.kernel,
        out_type=jax.ShapeDtypeStruct((n_tokens, EMBED), jnp.float32),
        mesh=mesh,
        compiler_params=pltpu.CompilerParams(use_tc_tiling_on_sc=False),
        scratch_types=[
            pltpu.VMEM((rows, CHUNK), jnp.int32),
            pltpu.VMEM((NBUF * CHUNK * FEAT,), jnp.float32),
            pltpu.VMEM((NBUF, CHUNK, EMBED), jnp.float32),
            pltpu.VMEM((FEAT, EMBED), jnp.float32),
            pltpu.VMEM((EMBED,), jnp.float32),
        ] + [pltpu.SemaphoreType.DMA] * (3 * NBUF),
    )
    def k(ids_hbm, feats_hbm, table_hbm, w_hbm, b_hbm, out_hbm,
          ids_v, feats_v, acc_v, w_v, b_v, *sems):
        gsem = sems[0:NBUF]
        fsem = sems[NBUF:2 * NBUF]
        osem = sems[2 * NBUF:3 * NBUF]
        wid = lax.axis_index("s") * 2 + lax.axis_index("c")
        tok0 = wid * per_worker

        pltpu.sync_copy(w_hbm, w_v)
        pltpu.sync_copy(b_hbm, b_v)
        pltpu.sync_copy(ids_hbm.at[pl.ds(wid * rows, rows)], ids_v)
        wv = [[w_v[r, pl.ds(LANES * j, LANES)] for j in range(NJ)]
              for r in range(FEAT)]
        bv = [b_v[pl.ds(LANES * j, LANES)] for j in range(NJ)]
        iota = lax.iota(jnp.int32, 16)

        FCH = CHUNK * FEAT

        def start_chunk(c, m):
            pltpu.async_copy(table_hbm.at[ids_v.at[c]], acc_v.at[m], gsem[m])
            pltpu.async_copy(feats_hbm.at[pl.ds((tok0 + c * CHUNK) * FEAT, FCH)],
                             feats_v.at[pl.ds(m * FCH, FCH)], fsem[m])

        def wait_chunk(c, m):
            pltpu.make_async_copy(
                table_hbm.at[ids_v.at[c]], acc_v.at[m], gsem[m]).wait()
            pltpu.make_async_copy(
                feats_hbm.at[pl.ds(tok0 * FEAT, FCH)],
                feats_v.at[pl.ds(m * FCH, FCH)], fsem[m]).wait()

        def compute(m):
            @pl.loop(0, CHUNK // 16)
            def _(g):
                fbase = m * (CHUNK * FEAT) + g * (16 * FEAT)
                vq = [feats_v[pl.ds(fbase + q * 16, 16)] for q in range(FEAT)]
                for i in range(16):
                    row = g * 16 + i
                    s = [vq[(FEAT * i + r) // 16][(FEAT * i + r) % 16]
                         for r in range(FEAT)]
                    for j in range(NJ):
                        p = bv[j] + s[0] * wv[0][j]
                        p = p + s[1] * wv[1][j]
                        p = p + s[2] * wv[2][j]
                        plsc.addupdate(
                            acc_v.at[m, row, pl.ds(LANES * j, LANES)], p)

        def start_scatter(c, m):
            dst = out_hbm.at[pl.ds(tok0 + c * CHUNK, CHUNK)]
            pltpu.async_copy(acc_v.at[m], dst, osem[m])

        def wait_scatter(m):
            pltpu.make_async_copy(
                acc_v.at[m], out_hbm.at[pl.ds(tok0, CHUNK)], osem[m]).wait()

        @pl.loop(0, kiters)
        def _(k_it):
            for m in range(NBUF):
                c = k_it * NBUF + m
                mp = (m - 1) % NBUF

                @pl.when(k_it > 0)
                def _():
                    wait_scatter(m)

                start_chunk(c, m)

                def fin():
                    wait_chunk(c - 1, mp)
                    compute(mp)
                    start_scatter(c - 1, mp)

                if m == 0:
                    @pl.when(k_it > 0)
                    def _():
                        fin()
                else:
                    fin()

        last = rows - 1
        wait_chunk(last, last % NBUF)
        compute(last % NBUF)
        start_scatter(last, last % NBUF)
        for m in range(NBUF):
            wait_scatter(m)

    return k


TBLK = 512


def _repack_table(table_t):
    v = table_t.shape[1]
    grid = (v + TBLK - 1) // TBLK

    def body(in_ref, out_ref):
        x = in_ref[...]
        out_ref[:, 0:EMBED] = jnp.transpose(x, (1, 0))

    return pl.pallas_call(
        body,
        grid=(grid,),
        in_specs=[pl.BlockSpec((EMBED, TBLK), lambda g: (0, g))],
        out_specs=pl.BlockSpec((TBLK, 128), lambda g: (g, 0)),
        out_shape=jax.ShapeDtypeStruct((v, 128), jnp.float32),
    )(table_t)


def kernel(ids, feats, table, W, b):
    bsz, seq = ids.shape
    n = bsz * seq
    nrows = table.shape[0]
    ids2d = (ids.astype(jnp.int32) * 2).reshape(n // CHUNK, CHUNK)
    featsf = feats.reshape(n * FEAT)
    padded = _repack_table(table.T)
    table_rm = padded.reshape(2 * nrows, EMBED)
    out = _sc_call(n)(ids2d, featsf, table_rm, W, b)
    return out.reshape(bsz, seq, EMBED)
